# baseline (speedup 1.0000x reference)
import jax
import jax.numpy as jnp
from jax.experimental import pallas as pl
from jax.experimental.pallas import tpu as pltpu

_L, _K, _D = 32, 1024, 2048
_LAMBDA = 0.1
_CC_LOW = 768
_CC_HIGH = 1024
_TOTAL_UNITS = 16 * _CC_LOW + 16 * _CC_HIGH
_ROWS = _L * _K
_BK = 256
_BPL = _K // _BK
_NVALID = 16 * 3 + 16 * 4


def _block_index(g):
    return jnp.where(g < 48, (g // 3) * 4 + g % 3, g + 16)


def _block_kernel(x_ref, out_ref):
    g = pl.program_id(0)

    @pl.when(g == 0)
    def _init():
        out_ref[0, 0] = jnp.float32(0.0)

    x = x_ref[...]
    row_sum = jnp.sum(x, axis=1)

    blk = _block_index(g)
    abs_row = blk * _BK + jax.lax.broadcasted_iota(jnp.int32, (_BK,), 0)
    k = jnp.bitwise_and(abs_row, _K - 1)
    col = jax.lax.broadcasted_iota(jnp.int32, (_BK, _D), 1)
    diag_mask = col == k[:, None]
    r_dk = jnp.sum(jnp.where(diag_mask, x, 0.0), axis=1)

    r_minus = (row_sum - r_dk) * jnp.float32(1.0 / (_D - 1))
    a = jnp.abs(r_dk)
    b = jnp.abs(r_minus)
    s = (a - b) / (a + b + jnp.float32(1e-6))
    out_ref[0, 0] += jnp.sum(s)


def kernel(R):
    flat = R.reshape(_ROWS, _D)
    total = pl.pallas_call(
        _block_kernel,
        grid=(_NVALID,),
        in_specs=[pl.BlockSpec((_BK, _D), lambda g: (_block_index(g), 0))],
        out_specs=pl.BlockSpec(
            (1, 1), lambda g: (0, 0), memory_space=pltpu.SMEM
        ),
        out_shape=jax.ShapeDtypeStruct((1, 1), jnp.float32),
    )(flat)
    return total[0, 0] * jnp.float32(-_LAMBDA / _TOTAL_UNITS)


# speedup vs baseline: 1.5421x; 1.5421x over previous
import jax
import jax.numpy as jnp
from jax.experimental import pallas as pl
from jax.experimental.pallas import tpu as pltpu

_L, _K, _D = 32, 1024, 2048
_LAMBDA = 0.1
_CC_LOW = 768
_CC_HIGH = 1024
_TOTAL_UNITS = 16 * _CC_LOW + 16 * _CC_HIGH


def _body(x, kk):
    row_sum = jnp.sum(x, axis=1)
    col = jax.lax.broadcasted_iota(jnp.int32, x.shape, 1)
    r_dk = jnp.sum(jnp.where(col == kk[:, None], x, 0.0), axis=1)
    r_minus = (row_sum - r_dk) * jnp.float32(1.0 / (_D - 1))
    a = jnp.abs(r_dk)
    b = jnp.abs(r_minus)
    return jnp.sum((a - b) / (a + b + jnp.float32(1e-6)))


def _low_kernel(x_ref, out_ref):
    @pl.when(pl.program_id(0) == 0)
    def _init():
        out_ref[0, 0] = jnp.float32(0.0)

    kk = jax.lax.broadcasted_iota(jnp.int32, (_CC_LOW,), 0)
    out_ref[0, 0] += _body(x_ref[0], kk)


def _high_kernel(part_ref, x_ref, out_ref):
    @pl.when(pl.program_id(0) == 0)
    def _init():
        out_ref[0, 0] = part_ref[0, 0]

    kk = jax.lax.broadcasted_iota(jnp.int32, (_CC_HIGH,), 0)
    out_ref[0, 0] += _body(x_ref[0], kk)


def kernel(R):
    part = pl.pallas_call(
        _low_kernel,
        grid=(16,),
        in_specs=[pl.BlockSpec((1, _CC_LOW, _D), lambda l: (l, 0, 0))],
        out_specs=pl.BlockSpec((1, 1), lambda l: (0, 0),
                               memory_space=pltpu.SMEM),
        out_shape=jax.ShapeDtypeStruct((1, 1), jnp.float32),
    )(R)
    total = pl.pallas_call(
        _high_kernel,
        grid=(16,),
        in_specs=[
            pl.BlockSpec(memory_space=pltpu.SMEM),
            pl.BlockSpec((1, _CC_HIGH, _D), lambda l: (l + 16, 0, 0)),
        ],
        out_specs=pl.BlockSpec((1, 1), lambda l: (0, 0),
                               memory_space=pltpu.SMEM),
        out_shape=jax.ShapeDtypeStruct((1, 1), jnp.float32),
    )(part, R)
    return total[0, 0] * jnp.float32(-_LAMBDA / _TOTAL_UNITS)


# speedup vs baseline: 1.6481x; 1.0688x over previous
import jax
import jax.numpy as jnp
from jax.experimental import pallas as pl
from jax.experimental.pallas import tpu as pltpu

_L, _K, _D = 32, 1024, 2048
_LAMBDA = 0.1
_CC_LOW = 768
_CC_HIGH = 1024
_TOTAL_UNITS = 16 * _CC_LOW + 16 * _CC_HIGH


def _body(x, kk):
    row_sum = jnp.sum(x, axis=1)
    col = jax.lax.broadcasted_iota(jnp.int32, x.shape, 1)
    r_dk = jnp.sum(jnp.where(col == kk[:, None], x, 0.0), axis=1)
    r_minus = (row_sum - r_dk) * jnp.float32(1.0 / (_D - 1))
    a = jnp.abs(r_dk)
    b = jnp.abs(r_minus)
    return jnp.sum((a - b) / (a + b + jnp.float32(1e-6)))


def _low_kernel(x_ref, out_ref):
    @pl.when(pl.program_id(0) == 0)
    def _init():
        out_ref[0, 0] = jnp.float32(0.0)

    r = jax.lax.broadcasted_iota(jnp.int32, (2 * _CC_LOW,), 0)
    kk = jnp.where(r >= _CC_LOW, r - _CC_LOW, r)
    x = x_ref[...].reshape(2 * _CC_LOW, _D)
    out_ref[0, 0] += _body(x, kk)


def _high_kernel(part_ref, x_ref, out_ref):
    @pl.when(pl.program_id(0) == 0)
    def _init():
        out_ref[0, 0] = part_ref[0, 0]

    kk = jnp.bitwise_and(
        jax.lax.broadcasted_iota(jnp.int32, (2 * _CC_HIGH,), 0), _K - 1
    )
    x = x_ref[...].reshape(2 * _CC_HIGH, _D)
    out_ref[0, 0] += _body(x, kk)


def kernel(R):
    part = pl.pallas_call(
        _low_kernel,
        grid=(8,),
        in_specs=[pl.BlockSpec((2, _CC_LOW, _D), lambda l: (l, 0, 0))],
        out_specs=pl.BlockSpec((1, 1), lambda l: (0, 0),
                               memory_space=pltpu.SMEM),
        out_shape=jax.ShapeDtypeStruct((1, 1), jnp.float32),
    )(R)
    total = pl.pallas_call(
        _high_kernel,
        grid=(8,),
        in_specs=[
            pl.BlockSpec(memory_space=pltpu.SMEM),
            pl.BlockSpec((2, _CC_HIGH, _D), lambda l: (l + 8, 0, 0)),
        ],
        out_specs=pl.BlockSpec((1, 1), lambda l: (0, 0),
                               memory_space=pltpu.SMEM),
        out_shape=jax.ShapeDtypeStruct((1, 1), jnp.float32),
    )(part, R)
    return total[0, 0] * jnp.float32(-_LAMBDA / _TOTAL_UNITS)
